# baseline (speedup 1.0000x reference)
import functools

import jax
import jax.numpy as jnp
from jax import lax
from jax.experimental import pallas as pl
from jax.experimental.pallas import tpu as pltpu
from jax.experimental.pallas import tpu_sc as plsc

_VOCAB = 50257
_ADDED_OFF = 50257
_VQ_START = 50769
_EMBED = 1024
_VQ_DIM = 256
_VQ_VOCAB = 8192
_N_ADDED = 512
_EXT_ROWS = _N_ADDED + _VQ_VOCAB

_NC, _NS, _LANES = 2, 16, 16
_NW = _NC * _NS
_TOKENS = 4 * 8192
_S_SC = 6144
_PER_W = _S_SC // _NW
_C = _LANES
_NCH = _PER_W // _C
_DUMMY = _S_SC
_SC_OUT_ROWS = _S_SC + 8
_B = 256
_SC_BLKS = _S_SC // _B


def _build_ext(W_add, W_cb, W_proj):
    def body(wadd_ref, wcb_ref, wproj_ref, out_ref):
        i = pl.program_id(0)

        @pl.when(i == 0)
        def _():
            out_ref[...] = wadd_ref[...]

        @pl.when(i > 0)
        def _():
            out_ref[...] = lax.dot_general(
                wcb_ref[...],
                wproj_ref[...],
                (((1,), (1,)), ((), ())),
                preferred_element_type=jnp.float32,
            )

    return pl.pallas_call(
        body,
        grid=(_EXT_ROWS // _N_ADDED,),
        in_specs=[
            pl.BlockSpec((_N_ADDED, _EMBED), lambda i: (0, 0)),
            pl.BlockSpec((_N_ADDED, _VQ_DIM), lambda i: (jnp.maximum(i - 1, 0), 0)),
            pl.BlockSpec((_EMBED, _VQ_DIM), lambda i: (0, 0)),
        ],
        out_specs=pl.BlockSpec((_N_ADDED, _EMBED), lambda i: (i, 0)),
        out_shape=jax.ShapeDtypeStruct((_EXT_ROWS, _EMBED), jnp.float32),
    )(W_add, W_cb, W_proj)


def _sc_lookup(x_sc, W_tok, W_ext):
    mesh = plsc.VectorSubcoreMesh(core_axis_name="c", subcore_axis_name="s")

    @functools.partial(
        pl.kernel,
        mesh=mesh,
        out_type=jax.ShapeDtypeStruct((_SC_OUT_ROWS, _EMBED), jnp.float32),
        scratch_types=[
            pltpu.VMEM((_PER_W,), jnp.int32),
            pltpu.VMEM((_C, _EMBED), jnp.float32),
            pltpu.VMEM((_C, _EMBED), jnp.float32),
            pltpu.VMEM((_C, _EMBED), jnp.float32),
            pltpu.VMEM((_C, _EMBED), jnp.float32),
            pltpu.SemaphoreType.DMA,
            pltpu.SemaphoreType.DMA,
            pltpu.SemaphoreType.DMA,
            pltpu.SemaphoreType.DMA,
            pltpu.SemaphoreType.DMA,
            pltpu.SemaphoreType.DMA,
            pltpu.SemaphoreType.DMA,
            pltpu.SemaphoreType.DMA,
        ],
    )
    def k(x_hbm, tok_hbm, ext_hbm, out_hbm, xv,
          ta0, ta1, eb0, eb1, gsa0, gsa1, wsa0, wsa1, gsb0, gsb1, wsb0, wsb1):
        wid = lax.axis_index("s") * _NC + lax.axis_index("c")
        base = pl.multiple_of(wid * _PER_W, _C)
        pltpu.sync_copy(x_hbm.at[pl.ds(base, _PER_W)], xv)

        lane = lax.iota(jnp.int32, _LANES)
        tbuf = (ta0, ta1)
        ebuf = (eb0, eb1)
        gsa = (gsa0, gsa1)
        wsa = (wsa0, wsa1)
        gsb = (gsb0, gsb1)
        wsb = (wsb0, wsb1)

        def chunk_idx(i):
            off = pl.multiple_of(i * _C, _C)
            v = xv[pl.ds(off, _C)]
            is_text = v < _ADDED_OFF
            pos = base + off + lane
            tok_i = jnp.where(is_text, v, 0)
            ext_i = jnp.where(is_text, 0, v - _ADDED_OFF)
            dst_t = jnp.where(is_text, pos, _DUMMY)
            dst_e = jnp.where(is_text, _DUMMY, pos)
            return tok_i, ext_i, dst_t, dst_e

        def start_gathers(i, s):
            tok_i, ext_i, _, _ = chunk_idx(i)
            pltpu.make_async_copy(tok_hbm.at[tok_i], tbuf[s], gsa[s]).start()
            pltpu.make_async_copy(ext_hbm.at[ext_i], ebuf[s], gsb[s]).start()

        def finish_chunk(i, s):
            _, _, dst_t, dst_e = chunk_idx(i)
            pltpu.make_async_copy(tok_hbm.at[lane], tbuf[s], gsa[s]).wait()
            pltpu.make_async_copy(tbuf[s], out_hbm.at[dst_t], wsa[s]).start()
            pltpu.make_async_copy(ext_hbm.at[lane], ebuf[s], gsb[s]).wait()
            pltpu.make_async_copy(ebuf[s], out_hbm.at[dst_e], wsb[s]).start()

        def drain_writes(i, s):
            _, _, dst_t, dst_e = chunk_idx(i)
            pltpu.make_async_copy(tbuf[s], out_hbm.at[dst_t], wsa[s]).wait()
            pltpu.make_async_copy(ebuf[s], out_hbm.at[dst_e], wsb[s]).wait()

        start_gathers(0, 0)

        def dma_body(g, carry):
            for b in range(2):
                i = 2 * g + b
                nxt = (b + 1) % 2

                @pl.when(i + 1 < _NCH)
                def _():
                    @pl.when(i >= 1)
                    def _():
                        drain_writes(i - 1, nxt)

                    start_gathers(i + 1, nxt)

                finish_chunk(i, b)
            return carry

        lax.fori_loop(0, _NCH // 2, dma_body, 0)
        drain_writes(_NCH - 2, 0)
        drain_writes(_NCH - 1, 1)

    return k(x_sc, W_tok, W_ext)


def _tc_gather(x_tc, W_tok, W_ext):
    n = x_tc.shape[0]

    n_blk = n // _B

    def body(ids_ref, tok_hbm, ext_hbm, out_hbm, sem):
        i = pl.program_id(0)
        blk = pl.multiple_of(i * _B, _B)

        def row(jj, carry):
            for u in range(4):
                j = jj * 4 + u
                v = ids_ref[j]

                @pl.when(v < _ADDED_OFF)
                def _():
                    pltpu.make_async_copy(
                        tok_hbm.at[pl.ds(v, 1)],
                        out_hbm.at[pl.ds(blk + j, 1)],
                        sem,
                    ).start()

                @pl.when(v >= _ADDED_OFF)
                def _():
                    pltpu.make_async_copy(
                        ext_hbm.at[pl.ds(v - _ADDED_OFF, 1)],
                        out_hbm.at[pl.ds(blk + j, 1)],
                        sem,
                    ).start()

            return carry

        lax.fori_loop(0, _B // 4, row, 0)

        @pl.when(i == n_blk - 1)
        def _():
            pltpu.make_async_copy(
                tok_hbm.at[pl.ds(0, n)], out_hbm, sem).wait()

    return pl.pallas_call(
        body,
        grid=(n_blk,),
        in_specs=[
            pl.BlockSpec((_B,), lambda i: (i,),
                         memory_space=pltpu.MemorySpace.SMEM),
            pl.BlockSpec(memory_space=pltpu.MemorySpace.HBM),
            pl.BlockSpec(memory_space=pltpu.MemorySpace.HBM),
        ],
        out_specs=pl.BlockSpec(memory_space=pltpu.MemorySpace.HBM),
        out_shape=jax.ShapeDtypeStruct((n, _EMBED), jnp.float32),
        scratch_shapes=[pltpu.SemaphoreType.DMA],
    )(x_tc, W_tok, W_ext)


def kernel(x, W_tok, W_add, W_cb, W_proj):
    W_ext = _build_ext(W_add, W_cb, W_proj)
    x_flat = x.reshape(-1)
    out = _tc_gather(x_flat, W_tok, W_ext)
    return out.reshape(x.shape + (_EMBED,))


# speedup vs baseline: 11.2827x; 11.2827x over previous
import functools

import jax
import jax.numpy as jnp
from jax import lax
from jax.experimental import pallas as pl
from jax.experimental.pallas import tpu as pltpu
from jax.experimental.pallas import tpu_sc as plsc

_VOCAB = 50257
_ADDED_OFF = 50257
_VQ_START = 50769
_EMBED = 1024
_VQ_DIM = 256
_VQ_VOCAB = 8192
_N_ADDED = 512
_EXT_ROWS = _N_ADDED + _VQ_VOCAB

_NC, _NS, _LANES = 2, 16, 16
_NW = _NC * _NS
_TOKENS = 4 * 8192
_S_SC = 6144
_PER_W = _S_SC // _NW
_C = _LANES
_NCH = _PER_W // _C
_DUMMY = _S_SC
_SC_OUT_ROWS = _S_SC + 8
_B = 512
_SC_BLKS = _S_SC // _B


def _build_ext(W_add, W_cb, W_proj):
    def body(wadd_ref, wcb_ref, wproj_ref, out_ref):
        i = pl.program_id(0)

        @pl.when(i == 0)
        def _():
            out_ref[...] = wadd_ref[...]

        @pl.when(i > 0)
        def _():
            out_ref[...] = lax.dot_general(
                wcb_ref[...],
                wproj_ref[...],
                (((1,), (1,)), ((), ())),
                preferred_element_type=jnp.float32,
            )

    return pl.pallas_call(
        body,
        grid=(_EXT_ROWS // _N_ADDED,),
        in_specs=[
            pl.BlockSpec((_N_ADDED, _EMBED), lambda i: (0, 0)),
            pl.BlockSpec((_N_ADDED, _VQ_DIM), lambda i: (jnp.maximum(i - 1, 0), 0)),
            pl.BlockSpec((_EMBED, _VQ_DIM), lambda i: (0, 0)),
        ],
        out_specs=pl.BlockSpec((_N_ADDED, _EMBED), lambda i: (i, 0)),
        out_shape=jax.ShapeDtypeStruct((_EXT_ROWS, _EMBED), jnp.float32),
    )(W_add, W_cb, W_proj)


def _sc_lookup(x_sc, W_tok, W_ext):
    mesh = plsc.VectorSubcoreMesh(core_axis_name="c", subcore_axis_name="s")

    @functools.partial(
        pl.kernel,
        mesh=mesh,
        out_type=jax.ShapeDtypeStruct((_SC_OUT_ROWS, _EMBED), jnp.float32),
        scratch_types=[
            pltpu.VMEM((_PER_W,), jnp.int32),
            pltpu.VMEM((_C, _EMBED), jnp.float32),
            pltpu.VMEM((_C, _EMBED), jnp.float32),
            pltpu.VMEM((_C, _EMBED), jnp.float32),
            pltpu.VMEM((_C, _EMBED), jnp.float32),
            pltpu.SemaphoreType.DMA,
            pltpu.SemaphoreType.DMA,
            pltpu.SemaphoreType.DMA,
            pltpu.SemaphoreType.DMA,
            pltpu.SemaphoreType.DMA,
            pltpu.SemaphoreType.DMA,
            pltpu.SemaphoreType.DMA,
            pltpu.SemaphoreType.DMA,
        ],
    )
    def k(x_hbm, tok_hbm, ext_hbm, out_hbm, xv,
          ta0, ta1, eb0, eb1, gsa0, gsa1, wsa0, wsa1, gsb0, gsb1, wsb0, wsb1):
        wid = lax.axis_index("s") * _NC + lax.axis_index("c")
        base = pl.multiple_of(wid * _PER_W, _C)
        pltpu.sync_copy(x_hbm.at[pl.ds(base, _PER_W)], xv)

        lane = lax.iota(jnp.int32, _LANES)
        tbuf = (ta0, ta1)
        ebuf = (eb0, eb1)
        gsa = (gsa0, gsa1)
        wsa = (wsa0, wsa1)
        gsb = (gsb0, gsb1)
        wsb = (wsb0, wsb1)

        def chunk_idx(i):
            off = pl.multiple_of(i * _C, _C)
            v = xv[pl.ds(off, _C)]
            is_text = v < _ADDED_OFF
            pos = base + off + lane
            tok_i = jnp.where(is_text, v, 0)
            ext_i = jnp.where(is_text, 0, v - _ADDED_OFF)
            dst_t = jnp.where(is_text, pos, _DUMMY)
            dst_e = jnp.where(is_text, _DUMMY, pos)
            return tok_i, ext_i, dst_t, dst_e

        def start_gathers(i, s):
            tok_i, ext_i, _, _ = chunk_idx(i)
            pltpu.make_async_copy(tok_hbm.at[tok_i], tbuf[s], gsa[s]).start()
            pltpu.make_async_copy(ext_hbm.at[ext_i], ebuf[s], gsb[s]).start()

        def finish_chunk(i, s):
            _, _, dst_t, dst_e = chunk_idx(i)
            pltpu.make_async_copy(tok_hbm.at[lane], tbuf[s], gsa[s]).wait()
            pltpu.make_async_copy(tbuf[s], out_hbm.at[dst_t], wsa[s]).start()
            pltpu.make_async_copy(ext_hbm.at[lane], ebuf[s], gsb[s]).wait()
            pltpu.make_async_copy(ebuf[s], out_hbm.at[dst_e], wsb[s]).start()

        def drain_writes(i, s):
            _, _, dst_t, dst_e = chunk_idx(i)
            pltpu.make_async_copy(tbuf[s], out_hbm.at[dst_t], wsa[s]).wait()
            pltpu.make_async_copy(ebuf[s], out_hbm.at[dst_e], wsb[s]).wait()

        start_gathers(0, 0)

        def dma_body(g, carry):
            for b in range(2):
                i = 2 * g + b
                nxt = (b + 1) % 2

                @pl.when(i + 1 < _NCH)
                def _():
                    @pl.when(i >= 1)
                    def _():
                        drain_writes(i - 1, nxt)

                    start_gathers(i + 1, nxt)

                finish_chunk(i, b)
            return carry

        lax.fori_loop(0, _NCH // 2, dma_body, 0)
        drain_writes(_NCH - 2, 0)
        drain_writes(_NCH - 1, 1)

    return k(x_sc, W_tok, W_ext)


def _tc_gather(x_tc, W_tok, W_ext):
    n = x_tc.shape[0]

    def body(ids_ref, tok_hbm, ext_hbm, out_ref, sem):
        def row(jj, carry):
            for u in range(8):
                j = jj * 8 + u
                v = ids_ref[j]

                @pl.when(v < _ADDED_OFF)
                def _():
                    pltpu.make_async_copy(
                        tok_hbm.at[pl.ds(v, 1)], out_ref.at[pl.ds(j, 1)], sem
                    ).start()

                @pl.when(v >= _ADDED_OFF)
                def _():
                    pltpu.make_async_copy(
                        ext_hbm.at[pl.ds(v - _ADDED_OFF, 1)],
                        out_ref.at[pl.ds(j, 1)],
                        sem,
                    ).start()

            return carry

        lax.fori_loop(0, _B // 8, row, 0)
        pltpu.make_async_copy(tok_hbm.at[pl.ds(0, _B)], out_ref, sem).wait()

    return pl.pallas_call(
        body,
        grid=(n // _B,),
        in_specs=[
            pl.BlockSpec((_B,), lambda i: (i,),
                         memory_space=pltpu.MemorySpace.SMEM),
            pl.BlockSpec(memory_space=pltpu.MemorySpace.HBM),
            pl.BlockSpec(memory_space=pltpu.MemorySpace.HBM),
        ],
        out_specs=pl.BlockSpec((_B, _EMBED), lambda i: (i, 0)),
        out_shape=jax.ShapeDtypeStruct((n, _EMBED), jnp.float32),
        scratch_shapes=[pltpu.SemaphoreType.DMA],
    )(x_tc, W_tok, W_ext)


def kernel(x, W_tok, W_add, W_cb, W_proj):
    W_ext = _build_ext(W_add, W_cb, W_proj)
    x_flat = x.reshape(-1)
    out = _tc_gather(x_flat, W_tok, W_ext)
    return out.reshape(x.shape + (_EMBED,))


# speedup vs baseline: 12.1420x; 1.0762x over previous
import functools

import jax
import jax.numpy as jnp
from jax import lax
from jax.experimental import pallas as pl
from jax.experimental.pallas import tpu as pltpu
from jax.experimental.pallas import tpu_sc as plsc

_VOCAB = 50257
_ADDED_OFF = 50257
_VQ_START = 50769
_EMBED = 1024
_VQ_DIM = 256
_VQ_VOCAB = 8192
_N_ADDED = 512
_EXT_ROWS = _N_ADDED + _VQ_VOCAB

_NC, _NS, _LANES = 2, 16, 16
_NW = _NC * _NS
_TOKENS = 4 * 8192
_S_SC = 6144
_PER_W = _S_SC // _NW
_C = _LANES
_NCH = _PER_W // _C
_DUMMY = _S_SC
_SC_OUT_ROWS = _S_SC + 8
_B = 1024
_SC_BLKS = _S_SC // _B


def _build_ext(W_add, W_cb, W_proj):
    def body(wadd_ref, wcb_ref, wproj_ref, out_ref):
        i = pl.program_id(0)

        @pl.when(i == 0)
        def _():
            out_ref[...] = wadd_ref[...]

        @pl.when(i > 0)
        def _():
            out_ref[...] = lax.dot_general(
                wcb_ref[...],
                wproj_ref[...],
                (((1,), (1,)), ((), ())),
                preferred_element_type=jnp.float32,
            )

    return pl.pallas_call(
        body,
        grid=(_EXT_ROWS // _N_ADDED,),
        in_specs=[
            pl.BlockSpec((_N_ADDED, _EMBED), lambda i: (0, 0)),
            pl.BlockSpec((_N_ADDED, _VQ_DIM), lambda i: (jnp.maximum(i - 1, 0), 0)),
            pl.BlockSpec((_EMBED, _VQ_DIM), lambda i: (0, 0)),
        ],
        out_specs=pl.BlockSpec((_N_ADDED, _EMBED), lambda i: (i, 0)),
        out_shape=jax.ShapeDtypeStruct((_EXT_ROWS, _EMBED), jnp.float32),
    )(W_add, W_cb, W_proj)


def _sc_lookup(x_sc, W_tok, W_ext):
    mesh = plsc.VectorSubcoreMesh(core_axis_name="c", subcore_axis_name="s")

    @functools.partial(
        pl.kernel,
        mesh=mesh,
        out_type=jax.ShapeDtypeStruct((_SC_OUT_ROWS, _EMBED), jnp.float32),
        scratch_types=[
            pltpu.VMEM((_PER_W,), jnp.int32),
            pltpu.VMEM((_C, _EMBED), jnp.float32),
            pltpu.VMEM((_C, _EMBED), jnp.float32),
            pltpu.VMEM((_C, _EMBED), jnp.float32),
            pltpu.VMEM((_C, _EMBED), jnp.float32),
            pltpu.SemaphoreType.DMA,
            pltpu.SemaphoreType.DMA,
            pltpu.SemaphoreType.DMA,
            pltpu.SemaphoreType.DMA,
            pltpu.SemaphoreType.DMA,
            pltpu.SemaphoreType.DMA,
            pltpu.SemaphoreType.DMA,
            pltpu.SemaphoreType.DMA,
        ],
    )
    def k(x_hbm, tok_hbm, ext_hbm, out_hbm, xv,
          ta0, ta1, eb0, eb1, gsa0, gsa1, wsa0, wsa1, gsb0, gsb1, wsb0, wsb1):
        wid = lax.axis_index("s") * _NC + lax.axis_index("c")
        base = pl.multiple_of(wid * _PER_W, _C)
        pltpu.sync_copy(x_hbm.at[pl.ds(base, _PER_W)], xv)

        lane = lax.iota(jnp.int32, _LANES)
        tbuf = (ta0, ta1)
        ebuf = (eb0, eb1)
        gsa = (gsa0, gsa1)
        wsa = (wsa0, wsa1)
        gsb = (gsb0, gsb1)
        wsb = (wsb0, wsb1)

        def chunk_idx(i):
            off = pl.multiple_of(i * _C, _C)
            v = xv[pl.ds(off, _C)]
            is_text = v < _ADDED_OFF
            pos = base + off + lane
            tok_i = jnp.where(is_text, v, 0)
            ext_i = jnp.where(is_text, 0, v - _ADDED_OFF)
            dst_t = jnp.where(is_text, pos, _DUMMY)
            dst_e = jnp.where(is_text, _DUMMY, pos)
            return tok_i, ext_i, dst_t, dst_e

        def start_gathers(i, s):
            tok_i, ext_i, _, _ = chunk_idx(i)
            pltpu.make_async_copy(tok_hbm.at[tok_i], tbuf[s], gsa[s]).start()
            pltpu.make_async_copy(ext_hbm.at[ext_i], ebuf[s], gsb[s]).start()

        def finish_chunk(i, s):
            _, _, dst_t, dst_e = chunk_idx(i)
            pltpu.make_async_copy(tok_hbm.at[lane], tbuf[s], gsa[s]).wait()
            pltpu.make_async_copy(tbuf[s], out_hbm.at[dst_t], wsa[s]).start()
            pltpu.make_async_copy(ext_hbm.at[lane], ebuf[s], gsb[s]).wait()
            pltpu.make_async_copy(ebuf[s], out_hbm.at[dst_e], wsb[s]).start()

        def drain_writes(i, s):
            _, _, dst_t, dst_e = chunk_idx(i)
            pltpu.make_async_copy(tbuf[s], out_hbm.at[dst_t], wsa[s]).wait()
            pltpu.make_async_copy(ebuf[s], out_hbm.at[dst_e], wsb[s]).wait()

        start_gathers(0, 0)

        def dma_body(g, carry):
            for b in range(2):
                i = 2 * g + b
                nxt = (b + 1) % 2

                @pl.when(i + 1 < _NCH)
                def _():
                    @pl.when(i >= 1)
                    def _():
                        drain_writes(i - 1, nxt)

                    start_gathers(i + 1, nxt)

                finish_chunk(i, b)
            return carry

        lax.fori_loop(0, _NCH // 2, dma_body, 0)
        drain_writes(_NCH - 2, 0)
        drain_writes(_NCH - 1, 1)

    return k(x_sc, W_tok, W_ext)


def _tc_gather(x_tc, W_tok, W_ext):
    n = x_tc.shape[0]

    def body(ids_ref, tok_hbm, ext_hbm, out_ref, sem):
        def row(jj, carry):
            for u in range(8):
                j = jj * 8 + u
                v = ids_ref[j]

                @pl.when(v < _ADDED_OFF)
                def _():
                    pltpu.make_async_copy(
                        tok_hbm.at[pl.ds(v, 1)], out_ref.at[pl.ds(j, 1)], sem
                    ).start()

                @pl.when(v >= _ADDED_OFF)
                def _():
                    pltpu.make_async_copy(
                        ext_hbm.at[pl.ds(v - _ADDED_OFF, 1)],
                        out_ref.at[pl.ds(j, 1)],
                        sem,
                    ).start()

            return carry

        lax.fori_loop(0, _B // 8, row, 0)
        pltpu.make_async_copy(tok_hbm.at[pl.ds(0, _B)], out_ref, sem).wait()

    return pl.pallas_call(
        body,
        grid=(n // _B,),
        in_specs=[
            pl.BlockSpec((_B,), lambda i: (i,),
                         memory_space=pltpu.MemorySpace.SMEM),
            pl.BlockSpec(memory_space=pltpu.MemorySpace.HBM),
            pl.BlockSpec(memory_space=pltpu.MemorySpace.HBM),
        ],
        out_specs=pl.BlockSpec((_B, _EMBED), lambda i: (i, 0)),
        out_shape=jax.ShapeDtypeStruct((n, _EMBED), jnp.float32),
        scratch_shapes=[pltpu.SemaphoreType.DMA],
    )(x_tc, W_tok, W_ext)


def kernel(x, W_tok, W_add, W_cb, W_proj):
    W_ext = _build_ext(W_add, W_cb, W_proj)
    x_flat = x.reshape(-1)
    out = _tc_gather(x_flat, W_tok, W_ext)
    return out.reshape(x.shape + (_EMBED,))


# speedup vs baseline: 12.5930x; 1.0371x over previous
import functools

import jax
import jax.numpy as jnp
from jax import lax
from jax.experimental import pallas as pl
from jax.experimental.pallas import tpu as pltpu
from jax.experimental.pallas import tpu_sc as plsc

_VOCAB = 50257
_ADDED_OFF = 50257
_VQ_START = 50769
_EMBED = 1024
_VQ_DIM = 256
_VQ_VOCAB = 8192
_N_ADDED = 512
_EXT_ROWS = _N_ADDED + _VQ_VOCAB

_NC, _NS, _LANES = 2, 16, 16
_NW = _NC * _NS
_TOKENS = 4 * 8192
_S_SC = 6144
_PER_W = _S_SC // _NW
_C = _LANES
_NCH = _PER_W // _C
_DUMMY = _S_SC
_SC_OUT_ROWS = _S_SC + 8
_B = 2048
_SC_BLKS = _S_SC // _B


def _build_ext(W_add, W_cb, W_proj):
    def body(wadd_ref, wcb_ref, wproj_ref, out_ref):
        i = pl.program_id(0)

        @pl.when(i == 0)
        def _():
            out_ref[...] = wadd_ref[...]

        @pl.when(i > 0)
        def _():
            out_ref[...] = lax.dot_general(
                wcb_ref[...],
                wproj_ref[...],
                (((1,), (1,)), ((), ())),
                preferred_element_type=jnp.float32,
            )

    return pl.pallas_call(
        body,
        grid=(_EXT_ROWS // _N_ADDED,),
        in_specs=[
            pl.BlockSpec((_N_ADDED, _EMBED), lambda i: (0, 0)),
            pl.BlockSpec((_N_ADDED, _VQ_DIM), lambda i: (jnp.maximum(i - 1, 0), 0)),
            pl.BlockSpec((_EMBED, _VQ_DIM), lambda i: (0, 0)),
        ],
        out_specs=pl.BlockSpec((_N_ADDED, _EMBED), lambda i: (i, 0)),
        out_shape=jax.ShapeDtypeStruct((_EXT_ROWS, _EMBED), jnp.float32),
    )(W_add, W_cb, W_proj)


def _sc_lookup(x_sc, W_tok, W_ext):
    mesh = plsc.VectorSubcoreMesh(core_axis_name="c", subcore_axis_name="s")

    @functools.partial(
        pl.kernel,
        mesh=mesh,
        out_type=jax.ShapeDtypeStruct((_SC_OUT_ROWS, _EMBED), jnp.float32),
        scratch_types=[
            pltpu.VMEM((_PER_W,), jnp.int32),
            pltpu.VMEM((_C, _EMBED), jnp.float32),
            pltpu.VMEM((_C, _EMBED), jnp.float32),
            pltpu.VMEM((_C, _EMBED), jnp.float32),
            pltpu.VMEM((_C, _EMBED), jnp.float32),
            pltpu.SemaphoreType.DMA,
            pltpu.SemaphoreType.DMA,
            pltpu.SemaphoreType.DMA,
            pltpu.SemaphoreType.DMA,
            pltpu.SemaphoreType.DMA,
            pltpu.SemaphoreType.DMA,
            pltpu.SemaphoreType.DMA,
            pltpu.SemaphoreType.DMA,
        ],
    )
    def k(x_hbm, tok_hbm, ext_hbm, out_hbm, xv,
          ta0, ta1, eb0, eb1, gsa0, gsa1, wsa0, wsa1, gsb0, gsb1, wsb0, wsb1):
        wid = lax.axis_index("s") * _NC + lax.axis_index("c")
        base = pl.multiple_of(wid * _PER_W, _C)
        pltpu.sync_copy(x_hbm.at[pl.ds(base, _PER_W)], xv)

        lane = lax.iota(jnp.int32, _LANES)
        tbuf = (ta0, ta1)
        ebuf = (eb0, eb1)
        gsa = (gsa0, gsa1)
        wsa = (wsa0, wsa1)
        gsb = (gsb0, gsb1)
        wsb = (wsb0, wsb1)

        def chunk_idx(i):
            off = pl.multiple_of(i * _C, _C)
            v = xv[pl.ds(off, _C)]
            is_text = v < _ADDED_OFF
            pos = base + off + lane
            tok_i = jnp.where(is_text, v, 0)
            ext_i = jnp.where(is_text, 0, v - _ADDED_OFF)
            dst_t = jnp.where(is_text, pos, _DUMMY)
            dst_e = jnp.where(is_text, _DUMMY, pos)
            return tok_i, ext_i, dst_t, dst_e

        def start_gathers(i, s):
            tok_i, ext_i, _, _ = chunk_idx(i)
            pltpu.make_async_copy(tok_hbm.at[tok_i], tbuf[s], gsa[s]).start()
            pltpu.make_async_copy(ext_hbm.at[ext_i], ebuf[s], gsb[s]).start()

        def finish_chunk(i, s):
            _, _, dst_t, dst_e = chunk_idx(i)
            pltpu.make_async_copy(tok_hbm.at[lane], tbuf[s], gsa[s]).wait()
            pltpu.make_async_copy(tbuf[s], out_hbm.at[dst_t], wsa[s]).start()
            pltpu.make_async_copy(ext_hbm.at[lane], ebuf[s], gsb[s]).wait()
            pltpu.make_async_copy(ebuf[s], out_hbm.at[dst_e], wsb[s]).start()

        def drain_writes(i, s):
            _, _, dst_t, dst_e = chunk_idx(i)
            pltpu.make_async_copy(tbuf[s], out_hbm.at[dst_t], wsa[s]).wait()
            pltpu.make_async_copy(ebuf[s], out_hbm.at[dst_e], wsb[s]).wait()

        start_gathers(0, 0)

        def dma_body(g, carry):
            for b in range(2):
                i = 2 * g + b
                nxt = (b + 1) % 2

                @pl.when(i + 1 < _NCH)
                def _():
                    @pl.when(i >= 1)
                    def _():
                        drain_writes(i - 1, nxt)

                    start_gathers(i + 1, nxt)

                finish_chunk(i, b)
            return carry

        lax.fori_loop(0, _NCH // 2, dma_body, 0)
        drain_writes(_NCH - 2, 0)
        drain_writes(_NCH - 1, 1)

    return k(x_sc, W_tok, W_ext)


def _tc_gather(x_tc, W_tok, W_ext):
    n = x_tc.shape[0]

    def body(ids_ref, tok_hbm, ext_hbm, out_ref, sem):
        def row(jj, carry):
            for u in range(8):
                j = jj * 8 + u
                v = ids_ref[j]

                @pl.when(v < _ADDED_OFF)
                def _():
                    pltpu.make_async_copy(
                        tok_hbm.at[pl.ds(v, 1)], out_ref.at[pl.ds(j, 1)], sem
                    ).start()

                @pl.when(v >= _ADDED_OFF)
                def _():
                    pltpu.make_async_copy(
                        ext_hbm.at[pl.ds(v - _ADDED_OFF, 1)],
                        out_ref.at[pl.ds(j, 1)],
                        sem,
                    ).start()

            return carry

        lax.fori_loop(0, _B // 8, row, 0)
        pltpu.make_async_copy(tok_hbm.at[pl.ds(0, _B)], out_ref, sem).wait()

    return pl.pallas_call(
        body,
        grid=(n // _B,),
        in_specs=[
            pl.BlockSpec((_B,), lambda i: (i,),
                         memory_space=pltpu.MemorySpace.SMEM),
            pl.BlockSpec(memory_space=pltpu.MemorySpace.HBM),
            pl.BlockSpec(memory_space=pltpu.MemorySpace.HBM),
        ],
        out_specs=pl.BlockSpec((_B, _EMBED), lambda i: (i, 0)),
        out_shape=jax.ShapeDtypeStruct((n, _EMBED), jnp.float32),
        scratch_shapes=[pltpu.SemaphoreType.DMA],
    )(x_tc, W_tok, W_ext)


def kernel(x, W_tok, W_add, W_cb, W_proj):
    W_ext = _build_ext(W_add, W_cb, W_proj)
    x_flat = x.reshape(-1)
    out = _tc_gather(x_flat, W_tok, W_ext)
    return out.reshape(x.shape + (_EMBED,))
